# baseline (speedup 1.0000x reference)
import jax
import jax.numpy as jnp
from jax import lax
from jax.experimental import pallas as pl
from jax.experimental.pallas import tpu as pltpu

B = 16384
ACTION_DIM = 256
H = 1024
D = 256
K = 8192
BETA = 0.25

GRID = 16
NSTREAM = 4
BLOCK_B = B // (NSTREAM * GRID)
CB_CHUNKS = 8
CB_ROWS = K // CB_CHUNKS


def _body(a_ref, a2_ref, a3_ref, a4_ref,
          b1_ref, b2_ref, bmu_ref, bd1_ref, bd2_ref, bo_ref,
          W1_hbm, W2_hbm, Wmu_hbm, cb_hbm, Wd1_hbm, Wd2_hbm, Wo_hbm,
          out_ref,
          w1_s, w2_s, wmu_s, cb_s, wd1_s, wd2_s, wo_s,
          x_s, colsum_s, sqsum_s, enc_s, q_s, d1_s, d2_s, r_s,
          minv_s, mini_s,
          sem_enc, sem_cb, sem_dec):
    i = pl.program_id(0)

    def enc_copies():
        return [pltpu.make_async_copy(W1_hbm, w1_s, sem_enc.at[0]),
                pltpu.make_async_copy(W2_hbm, w2_s, sem_enc.at[1]),
                pltpu.make_async_copy(Wmu_hbm, wmu_s, sem_enc.at[2])]

    def cb_copy(c):
        sl = pl.ds(c * CB_ROWS, CB_ROWS)
        return pltpu.make_async_copy(cb_hbm.at[sl, :], cb_s.at[sl, :],
                                     sem_cb.at[c])

    def dec_copies():
        return [pltpu.make_async_copy(Wd1_hbm, wd1_s, sem_dec.at[0]),
                pltpu.make_async_copy(Wd2_hbm, wd2_s, sem_dec.at[1]),
                pltpu.make_async_copy(Wo_hbm, wo_s, sem_dec.at[2])]

    @pl.when(i == 0)
    def _init():
        x_s[...] = a_ref[0:1, :]
        colsum_s[...] = jnp.zeros_like(colsum_s)
        sqsum_s[...] = jnp.zeros_like(sqsum_s)
        minv_s[0, 0] = jnp.inf
        mini_s[0, 0] = 0
        for cp in enc_copies():
            cp.start()
        for c in range(CB_CHUNKS):
            cb_copy(c).start()
        for cp in dec_copies():
            cp.start()

    parts = [r[...] for r in (a_ref, a2_ref, a3_ref, a4_ref)]
    colsum_s[...] += sum(jnp.sum(p, axis=0, keepdims=True) for p in parts)
    sqsum_s[...] += sum(jnp.sum(p * p, axis=0, keepdims=True) for p in parts)

    @pl.when(i == 2)
    def _encode():
        for cp in enc_copies():
            cp.wait()
        x = x_s[...]
        h1 = jnp.maximum(
            jnp.dot(x, w1_s[...], preferred_element_type=jnp.float32)
            + b1_ref[...], 0.0)
        h2 = jnp.maximum(
            jnp.dot(h1, w2_s[...], preferred_element_type=jnp.float32)
            + b2_ref[...], 0.0)
        enc_s[...] = (jnp.dot(h2, wmu_s[...],
                              preferred_element_type=jnp.float32)
                      + bmu_ref[...])

    for c in range(CB_CHUNKS):
        @pl.when(i == 3 + c)
        def _chunk(c=c):
            cb_copy(c).wait()
            cb = cb_s[c * CB_ROWS:(c + 1) * CB_ROWS, :]
            enc = enc_s[...]
            cb2 = jnp.sum(cb * cb, axis=1, keepdims=True)
            scores = lax.dot_general(cb, enc, (((1,), (1,)), ((), ())),
                                     preferred_element_type=jnp.float32)
            dist = cb2 - 2.0 * scores
            m = jnp.min(dist)
            ids = lax.broadcasted_iota(jnp.int32, (CB_ROWS, 1), 0) \
                + jnp.int32(c * CB_ROWS)
            idxc = jnp.min(jnp.where(dist == m, ids, jnp.int32(K)))
            better = m < minv_s[0, 0]
            mini_s[0, 0] = jnp.where(better, idxc, mini_s[0, 0])
            minv_s[0, 0] = jnp.where(better, m, minv_s[0, 0])

    @pl.when(i == 3 + CB_CHUNKS)
    def _gather_q():
        idx = mini_s[0, 0]
        q_s[...] = cb_s[pl.ds(idx, 1), :]

    @pl.when(i == 4 + CB_CHUNKS)
    def _dec1():
        for cp in dec_copies():
            cp.wait()
        d1_s[...] = jnp.maximum(
            jnp.dot(q_s[...], wd1_s[...], preferred_element_type=jnp.float32)
            + bd1_ref[...], 0.0)

    @pl.when(i == 5 + CB_CHUNKS)
    def _dec2():
        d2_s[...] = jnp.maximum(
            jnp.dot(d1_s[...], wd2_s[...], preferred_element_type=jnp.float32)
            + bd2_ref[...], 0.0)

    @pl.when(i == 6 + CB_CHUNKS)
    def _dec3():
        r_s[...] = jnp.tanh(
            jnp.dot(d2_s[...], wo_s[...], preferred_element_type=jnp.float32)
            + bo_ref[...])

    @pl.when(i == pl.num_programs(0) - 1)
    def _finish():
        enc = enc_s[...]
        q = q_s[...]
        mse_vq = jnp.mean((q - enc) ** 2)
        vq_loss = (1.0 + BETA) * mse_vq

        r = r_s[...]
        colsum = colsum_s[...]
        ss = jnp.sum(sqsum_s[...])
        bf = jnp.float32(B)
        recons = (bf * jnp.sum(r * r) - 2.0 * jnp.sum(r * colsum) + ss) \
            / (bf * jnp.float32(ACTION_DIM))
        total = recons + vq_loss

        lanes = lax.broadcasted_iota(jnp.int32, (8, 128), 1)
        out_ref[...] = jnp.where(
            lanes == 0, total,
            jnp.where(lanes == 1, recons,
                      jnp.where(lanes == 2, vq_loss, 0.0)))


def kernel(action, W_enc1, b_enc1, W_enc2, b_enc2, W_mu, b_mu, codebook,
           W_dec1, b_dec1, W_dec2, b_dec2, W_out, b_out):
    small = lambda shape: pl.BlockSpec(shape, lambda i: (0, 0))
    hbm = pl.BlockSpec(memory_space=pl.ANY)
    res = pl.pallas_call(
        _body,
        grid=(GRID,),
        in_specs=[
            pl.BlockSpec((BLOCK_B, ACTION_DIM), lambda i: (i, 0)),
            pl.BlockSpec((BLOCK_B, ACTION_DIM), lambda i: (GRID + i, 0)),
            pl.BlockSpec((BLOCK_B, ACTION_DIM), lambda i: (2 * GRID + i, 0)),
            pl.BlockSpec((BLOCK_B, ACTION_DIM), lambda i: (3 * GRID + i, 0)),
            small((1, H)), small((1, H)), small((1, D)),
            small((1, H)), small((1, H)), small((1, ACTION_DIM)),
            hbm, hbm, hbm, hbm, hbm, hbm, hbm,
        ],
        out_specs=pl.BlockSpec((8, 128), lambda i: (0, 0)),
        out_shape=jax.ShapeDtypeStruct((8, 128), jnp.float32),
        scratch_shapes=[
            pltpu.VMEM((ACTION_DIM, H), jnp.float32),
            pltpu.VMEM((H, H), jnp.float32),
            pltpu.VMEM((H, D), jnp.float32),
            pltpu.VMEM((K, D), jnp.float32),
            pltpu.VMEM((D, H), jnp.float32),
            pltpu.VMEM((H, H), jnp.float32),
            pltpu.VMEM((H, ACTION_DIM), jnp.float32),
            pltpu.VMEM((1, ACTION_DIM), jnp.float32),
            pltpu.VMEM((1, ACTION_DIM), jnp.float32),
            pltpu.VMEM((1, ACTION_DIM), jnp.float32),
            pltpu.VMEM((1, D), jnp.float32),
            pltpu.VMEM((1, D), jnp.float32),
            pltpu.VMEM((1, H), jnp.float32),
            pltpu.VMEM((1, H), jnp.float32),
            pltpu.VMEM((1, ACTION_DIM), jnp.float32),
            pltpu.SMEM((1, 1), jnp.float32),
            pltpu.SMEM((1, 1), jnp.int32),
            pltpu.SemaphoreType.DMA((3,)),
            pltpu.SemaphoreType.DMA((CB_CHUNKS,)),
            pltpu.SemaphoreType.DMA((3,)),
        ],
        compiler_params=pltpu.CompilerParams(
            vmem_limit_bytes=100 * 1024 * 1024,
        ),
    )(action, action, action, action,
      b_enc1.reshape(1, H), b_enc2.reshape(1, H), b_mu.reshape(1, D),
      b_dec1.reshape(1, H), b_dec2.reshape(1, H), b_out.reshape(1, ACTION_DIM),
      W_enc1, W_enc2, W_mu, codebook, W_dec1, W_dec2, W_out)
    return (res[0, 0], res[0, 1], res[0, 2])


# speedup vs baseline: 1.0474x; 1.0474x over previous
import jax
import jax.numpy as jnp
from jax import lax
from jax.experimental import pallas as pl
from jax.experimental.pallas import tpu as pltpu

B = 16384
ACTION_DIM = 256
H = 1024
D = 256
K = 8192
BETA = 0.25

GRID = 16
NSTREAM = 4
BLOCK_B = B // (NSTREAM * GRID)
CB_CHUNKS = 8
CB_ROWS = K // CB_CHUNKS


def _body(a_ref, a2_ref, a3_ref, a4_ref,
          b1_ref, b2_ref, bmu_ref, bd1_ref, bd2_ref, bo_ref,
          W1_hbm, W2_hbm, Wmu_hbm, cb_hbm, Wd1_hbm, Wd2_hbm, Wo_hbm,
          out_total, out_recons, out_vq,
          w1_s, w2_s, wmu_s, cb_s, wd1_s, wd2_s, wo_s,
          x_s, colsum_s, sqsum_s, enc_s, q_s, d1_s, d2_s, r_s,
          minv_s, mini_s,
          sem_enc, sem_cb, sem_dec):
    i = pl.program_id(0)

    def enc_copies():
        return [pltpu.make_async_copy(W1_hbm, w1_s, sem_enc.at[0]),
                pltpu.make_async_copy(W2_hbm, w2_s, sem_enc.at[1]),
                pltpu.make_async_copy(Wmu_hbm, wmu_s, sem_enc.at[2])]

    def cb_copy(c):
        sl = pl.ds(c * CB_ROWS, CB_ROWS)
        return pltpu.make_async_copy(cb_hbm.at[sl, :], cb_s.at[sl, :],
                                     sem_cb.at[c])

    def dec_copies():
        return [pltpu.make_async_copy(Wd1_hbm, wd1_s, sem_dec.at[0]),
                pltpu.make_async_copy(Wd2_hbm, wd2_s, sem_dec.at[1]),
                pltpu.make_async_copy(Wo_hbm, wo_s, sem_dec.at[2])]

    @pl.when(i == 0)
    def _init():
        x_s[...] = a_ref[0:1, :]
        colsum_s[...] = jnp.zeros_like(colsum_s)
        sqsum_s[...] = jnp.zeros_like(sqsum_s)
        minv_s[0, 0] = jnp.inf
        mini_s[0, 0] = 0
        for cp in enc_copies():
            cp.start()
        for c in range(CB_CHUNKS):
            cb_copy(c).start()
        for cp in dec_copies():
            cp.start()

    parts = [r[...] for r in (a_ref, a2_ref, a3_ref, a4_ref)]
    colsum_s[...] += sum(jnp.sum(p, axis=0, keepdims=True) for p in parts)
    sqsum_s[...] += sum(jnp.sum(p * p, axis=0, keepdims=True) for p in parts)

    @pl.when(i == 2)
    def _encode():
        for cp in enc_copies():
            cp.wait()
        x = x_s[...]
        h1 = jnp.maximum(
            jnp.dot(x, w1_s[...], preferred_element_type=jnp.float32)
            + b1_ref[...], 0.0)
        h2 = jnp.maximum(
            jnp.dot(h1, w2_s[...], preferred_element_type=jnp.float32)
            + b2_ref[...], 0.0)
        enc_s[...] = (jnp.dot(h2, wmu_s[...],
                              preferred_element_type=jnp.float32)
                      + bmu_ref[...])

    for c in range(CB_CHUNKS):
        @pl.when(i == 3 + c)
        def _chunk(c=c):
            cb_copy(c).wait()
            cb = cb_s[c * CB_ROWS:(c + 1) * CB_ROWS, :]
            enc = enc_s[...]
            cb2 = jnp.sum(cb * cb, axis=1, keepdims=True)
            scores = lax.dot_general(cb, enc, (((1,), (1,)), ((), ())),
                                     preferred_element_type=jnp.float32)
            dist = cb2 - 2.0 * scores
            m = jnp.min(dist)
            ids = lax.broadcasted_iota(jnp.int32, (CB_ROWS, 1), 0) \
                + jnp.int32(c * CB_ROWS)
            idxc = jnp.min(jnp.where(dist == m, ids, jnp.int32(K)))
            better = m < minv_s[0, 0]
            mini_s[0, 0] = jnp.where(better, idxc, mini_s[0, 0])
            minv_s[0, 0] = jnp.where(better, m, minv_s[0, 0])

    @pl.when(i == 3 + CB_CHUNKS)
    def _gather_q():
        idx = mini_s[0, 0]
        q_s[...] = cb_s[pl.ds(idx, 1), :]

    @pl.when(i == 4 + CB_CHUNKS)
    def _dec1():
        for cp in dec_copies():
            cp.wait()
        d1_s[...] = jnp.maximum(
            jnp.dot(q_s[...], wd1_s[...], preferred_element_type=jnp.float32)
            + bd1_ref[...], 0.0)

    @pl.when(i == 5 + CB_CHUNKS)
    def _dec2():
        d2_s[...] = jnp.maximum(
            jnp.dot(d1_s[...], wd2_s[...], preferred_element_type=jnp.float32)
            + bd2_ref[...], 0.0)

    @pl.when(i == 6 + CB_CHUNKS)
    def _dec3():
        r_s[...] = jnp.tanh(
            jnp.dot(d2_s[...], wo_s[...], preferred_element_type=jnp.float32)
            + bo_ref[...])

    @pl.when(i == pl.num_programs(0) - 1)
    def _finish():
        enc = enc_s[...]
        q = q_s[...]
        mse_vq = jnp.mean((q - enc) ** 2)
        vq_loss = (1.0 + BETA) * mse_vq

        r = r_s[...]
        colsum = colsum_s[...]
        ss = jnp.sum(sqsum_s[...])
        bf = jnp.float32(B)
        recons = (bf * jnp.sum(r * r) - 2.0 * jnp.sum(r * colsum) + ss) \
            / (bf * jnp.float32(ACTION_DIM))
        total = recons + vq_loss

        out_total[0, 0] = total
        out_recons[0, 0] = recons
        out_vq[0, 0] = vq_loss


def kernel(action, W_enc1, b_enc1, W_enc2, b_enc2, W_mu, b_mu, codebook,
           W_dec1, b_dec1, W_dec2, b_dec2, W_out, b_out):
    small = lambda shape: pl.BlockSpec(shape, lambda i: (0, 0))
    hbm = pl.BlockSpec(memory_space=pl.ANY)
    res = pl.pallas_call(
        _body,
        grid=(GRID,),
        in_specs=[
            pl.BlockSpec((BLOCK_B, ACTION_DIM), lambda i: (i, 0)),
            pl.BlockSpec((BLOCK_B, ACTION_DIM), lambda i: (GRID + i, 0)),
            pl.BlockSpec((BLOCK_B, ACTION_DIM), lambda i: (2 * GRID + i, 0)),
            pl.BlockSpec((BLOCK_B, ACTION_DIM), lambda i: (3 * GRID + i, 0)),
            small((1, H)), small((1, H)), small((1, D)),
            small((1, H)), small((1, H)), small((1, ACTION_DIM)),
            hbm, hbm, hbm, hbm, hbm, hbm, hbm,
        ],
        out_specs=[
            pl.BlockSpec(memory_space=pltpu.SMEM),
            pl.BlockSpec(memory_space=pltpu.SMEM),
            pl.BlockSpec(memory_space=pltpu.SMEM),
        ],
        out_shape=[
            jax.ShapeDtypeStruct((1, 1), jnp.float32),
            jax.ShapeDtypeStruct((1, 1), jnp.float32),
            jax.ShapeDtypeStruct((1, 1), jnp.float32),
        ],
        scratch_shapes=[
            pltpu.VMEM((ACTION_DIM, H), jnp.float32),
            pltpu.VMEM((H, H), jnp.float32),
            pltpu.VMEM((H, D), jnp.float32),
            pltpu.VMEM((K, D), jnp.float32),
            pltpu.VMEM((D, H), jnp.float32),
            pltpu.VMEM((H, H), jnp.float32),
            pltpu.VMEM((H, ACTION_DIM), jnp.float32),
            pltpu.VMEM((1, ACTION_DIM), jnp.float32),
            pltpu.VMEM((1, ACTION_DIM), jnp.float32),
            pltpu.VMEM((1, ACTION_DIM), jnp.float32),
            pltpu.VMEM((1, D), jnp.float32),
            pltpu.VMEM((1, D), jnp.float32),
            pltpu.VMEM((1, H), jnp.float32),
            pltpu.VMEM((1, H), jnp.float32),
            pltpu.VMEM((1, ACTION_DIM), jnp.float32),
            pltpu.SMEM((1, 1), jnp.float32),
            pltpu.SMEM((1, 1), jnp.int32),
            pltpu.SemaphoreType.DMA((3,)),
            pltpu.SemaphoreType.DMA((CB_CHUNKS,)),
            pltpu.SemaphoreType.DMA((3,)),
        ],
        compiler_params=pltpu.CompilerParams(
            vmem_limit_bytes=100 * 1024 * 1024,
        ),
    )(action, action, action, action,
      b_enc1.reshape(1, H), b_enc2.reshape(1, H), b_mu.reshape(1, D),
      b_dec1.reshape(1, H), b_dec2.reshape(1, H), b_out.reshape(1, ACTION_DIM),
      W_enc1, W_enc2, W_mu, codebook, W_dec1, W_dec2, W_out)
    return (res[0].reshape(()), res[1].reshape(()), res[2].reshape(()))
